# baseline (speedup 1.0000x reference)
import functools

import jax
import jax.numpy as jnp
from jax import lax
from jax.experimental import pallas as pl
from jax.experimental.pallas import tpu as pltpu
from jax.experimental.pallas import tpu_sc as plsc

L = 16
N_WORKERS = 32
TCHUNK = 128


def _tokenizer_sc(idx_t, pi_flat, emb_flat, ord_flat, *, b_total, t_len):
    d = idx_t.shape[0]
    vocab = emb_flat.shape[0]
    b_per_w = b_total // N_WORKERS
    b_rem = b_total - b_per_w * N_WORKERS
    halves = t_len // TCHUNK
    jparts = TCHUNK // L

    mesh = plsc.VectorSubcoreMesh(core_axis_name="c", subcore_axis_name="s")

    @functools.partial(
        pl.kernel,
        mesh=mesh,
        out_type=jax.ShapeDtypeStruct((b_total, d, t_len), jnp.float32),
        compiler_params=pltpu.CompilerParams(needs_layout_passes=False),
        scratch_types=[
            pltpu.VMEM((vocab,), jnp.float32),
            pltpu.VMEM((2 * d,), jnp.float32),
            pltpu.VMEM((d,), jnp.float32),
            pltpu.VMEM((d,), jnp.float32),
            pltpu.VMEM(((b_per_w + b_rem) * 2 * t_len,), jnp.int32),
            pltpu.VMEM((d, TCHUNK), jnp.int32),
            pltpu.VMEM((d, TCHUNK), jnp.int32),
            pltpu.VMEM((d, TCHUNK), jnp.float32),
            pltpu.VMEM((d, TCHUNK), jnp.float32),
            pltpu.SemaphoreType.DMA,
            pltpu.SemaphoreType.DMA,
            pltpu.SemaphoreType.DMA,
            pltpu.SemaphoreType.DMA,
        ],
    )
    def k(idx_hbm, pi_hbm, emb_hbm, ord_hbm, out_hbm,
          emb_v, ord_v, ord0_v, dord_v, pi_v,
          idx0, idx1, o0, o1, si0, si1, so0, so1):
        wid = lax.axis_index("s") * 2 + lax.axis_index("c")
        b0 = wid * b_per_w
        is_last = wid == N_WORKERS - 1
        nq = jnp.where(is_last, (b_per_w + b_rem) * halves, b_per_w * halves)

        pltpu.sync_copy(emb_hbm, emb_v)
        pltpu.sync_copy(ord_hbm, ord_v)
        pltpu.sync_copy(pi_hbm.at[pl.ds(b0 * 2 * t_len, b_per_w * 2 * t_len)],
                        pi_v.at[pl.ds(0, b_per_w * 2 * t_len)])

        @pl.when(is_last)
        def _():
            pltpu.sync_copy(
                pi_hbm.at[pl.ds((b0 + b_per_w) * 2 * t_len, b_rem * 2 * t_len)],
                pi_v.at[pl.ds(b_per_w * 2 * t_len, b_rem * 2 * t_len)])

        @plsc.parallel_loop(0, d // L)
        def ord_body(j):
            s = pl.ds(j * L, L)
            o0v = ord_v[s]
            ord0_v[s] = o0v
            dord_v[s] = ord_v[pl.ds(d + j * L, L)] - o0v

        idx_bufs = (idx0, idx1)
        out_bufs = (o0, o1)
        sin = (si0, si1)
        sout = (so0, so1)

        def in_copy(q, slot):
            col0 = (b0 + q // halves) * t_len + (q % halves) * TCHUNK
            return pltpu.make_async_copy(
                idx_hbm.at[:, pl.ds(col0, TCHUNK)], idx_bufs[slot], sin[slot])

        def out_copy(q, slot):
            return pltpu.make_async_copy(
                out_bufs[slot],
                out_hbm.at[b0 + q // halves, :,
                           pl.ds((q % halves) * TCHUNK, TCHUNK)],
                sout[slot])

        in_copy(0, 0).start()
        in_copy(1, 1).start()

        def compute(q, slot):
            ib = idx_bufs[slot]
            outb = out_bufs[slot]
            pi_base = (q // halves) * 2 * t_len + (q % halves) * TCHUNK

            obc = []
            for j in range(jparts):
                pav = pi_v[pl.ds(pi_base + j * L, L)]
                pbv = pi_v[pl.ds(pi_base + t_len + j * L, L)]
                obc.append(jnp.where(pav == pbv, 1.0, 0.0))

            @plsc.parallel_loop(0, d, unroll=8)
            def dd_body(dd):
                dsel = jnp.full((L,), dd, jnp.int32)
                ord0b = plsc.load_gather(ord0_v, [dsel])
                dordb = plsc.load_gather(dord_v, [dsel])
                for j in range(jparts):
                    s = pl.ds(j * L, L)
                    ev = plsc.load_gather(emb_v, [ib[dd, s]])
                    outb[dd, s] = ev + (ord0b + obc[j] * dordb)

        def pair_body(q2, _):
            for slot in range(2):
                q = q2 * 2 + slot

                in_copy(q, slot).wait()

                @pl.when(q2 > 0)
                def _():
                    out_copy(q - 2, slot).wait()

                compute(q, slot)
                out_copy(q, slot).start()

                @pl.when(q2 < nq // 2 - 1)
                def _():
                    in_copy(q + 2, slot).start()

            return 0

        lax.fori_loop(0, nq // 2, pair_body, 0)
        out_copy(nq - 2, 0).wait()
        out_copy(nq - 1, 1).wait()

    return k(idx_t, pi_flat, emb_flat, ord_flat)


def kernel(edge_index, edge_data, node_data, node_num, edge_num, padded_index,
           padding_mask, padded_node_mask, padded_edge_mask, emb_table,
           order_table):
    b, t = padded_node_mask.shape
    d = node_data.shape[-1]

    idx_t = node_data.T
    pi_flat = padded_index.transpose(0, 2, 1).reshape(-1)
    emb_flat = emb_table.reshape(-1)
    ord_flat = order_table.reshape(-1)

    out3 = _tokenizer_sc(idx_t, pi_flat, emb_flat, ord_flat,
                         b_total=b, t_len=t)
    return out3.transpose(0, 2, 1)


# speedup vs baseline: 1.1082x; 1.1082x over previous
import functools

import jax
import jax.numpy as jnp
from jax import lax
from jax.experimental import pallas as pl
from jax.experimental.pallas import tpu as pltpu
from jax.experimental.pallas import tpu_sc as plsc

L = 16
N_WORKERS = 32
TCHUNK = 256


def _tokenizer_sc(idx_t, pi_flat, emb_flat, ord_flat, *, b_total, t_len):
    d = idx_t.shape[0]
    vocab = emb_flat.shape[0]
    b_per_w = b_total // N_WORKERS
    b_rem = b_total - b_per_w * N_WORKERS
    halves = t_len // TCHUNK
    jparts = TCHUNK // L

    mesh = plsc.VectorSubcoreMesh(core_axis_name="c", subcore_axis_name="s")

    @functools.partial(
        pl.kernel,
        mesh=mesh,
        out_type=jax.ShapeDtypeStruct((b_total, d, t_len), jnp.float32),
        compiler_params=pltpu.CompilerParams(needs_layout_passes=False),
        scratch_types=[
            pltpu.VMEM((vocab,), jnp.float32),
            pltpu.VMEM((2 * d,), jnp.float32),
            pltpu.VMEM((d,), jnp.float32),
            pltpu.VMEM((d,), jnp.float32),
            pltpu.VMEM(((b_per_w + b_rem) * 2 * t_len,), jnp.int32),
            pltpu.VMEM((d, TCHUNK), jnp.int32),
            pltpu.VMEM((d, TCHUNK), jnp.int32),
            pltpu.VMEM((d, TCHUNK), jnp.float32),
            pltpu.VMEM((d, TCHUNK), jnp.float32),
            pltpu.SemaphoreType.DMA,
            pltpu.SemaphoreType.DMA,
            pltpu.SemaphoreType.DMA,
            pltpu.SemaphoreType.DMA,
        ],
    )
    def k(idx_hbm, pi_hbm, emb_hbm, ord_hbm, out_hbm,
          emb_v, ord_v, ord0_v, dord_v, pi_v,
          idx0, idx1, o0, o1, si0, si1, so0, so1):
        wid = lax.axis_index("s") * 2 + lax.axis_index("c")
        b0 = wid * b_per_w
        is_last = wid == N_WORKERS - 1
        nq = jnp.where(is_last, (b_per_w + b_rem) * halves, b_per_w * halves)

        pltpu.sync_copy(emb_hbm, emb_v)
        pltpu.sync_copy(ord_hbm, ord_v)
        pltpu.sync_copy(pi_hbm.at[pl.ds(b0 * 2 * t_len, b_per_w * 2 * t_len)],
                        pi_v.at[pl.ds(0, b_per_w * 2 * t_len)])

        @pl.when(is_last)
        def _():
            pltpu.sync_copy(
                pi_hbm.at[pl.ds((b0 + b_per_w) * 2 * t_len, b_rem * 2 * t_len)],
                pi_v.at[pl.ds(b_per_w * 2 * t_len, b_rem * 2 * t_len)])

        @plsc.parallel_loop(0, d // L)
        def ord_body(j):
            s = pl.ds(j * L, L)
            o0v = ord_v[s]
            ord0_v[s] = o0v
            dord_v[s] = ord_v[pl.ds(d + j * L, L)] - o0v

        idx_bufs = (idx0, idx1)
        out_bufs = (o0, o1)
        sin = (si0, si1)
        sout = (so0, so1)

        def in_copy(q, slot):
            col0 = (b0 + q // halves) * t_len + (q % halves) * TCHUNK
            return pltpu.make_async_copy(
                idx_hbm.at[:, pl.ds(col0, TCHUNK)], idx_bufs[slot], sin[slot])

        def out_copy(q, slot):
            return pltpu.make_async_copy(
                out_bufs[slot],
                out_hbm.at[b0 + q // halves, :,
                           pl.ds((q % halves) * TCHUNK, TCHUNK)],
                sout[slot])

        in_copy(0, 0).start()
        in_copy(1, 1).start()

        def compute(q, slot):
            ib = idx_bufs[slot]
            outb = out_bufs[slot]
            pi_base = (q // halves) * 2 * t_len + (q % halves) * TCHUNK

            obc = []
            for j in range(jparts):
                pav = pi_v[pl.ds(pi_base + j * L, L)]
                pbv = pi_v[pl.ds(pi_base + t_len + j * L, L)]
                obc.append(jnp.where(pav == pbv, 1.0, 0.0))

            @plsc.parallel_loop(0, d, unroll=8)
            def dd_body(dd):
                dsel = jnp.full((L,), dd, jnp.int32)
                ord0b = plsc.load_gather(ord0_v, [dsel])
                dordb = plsc.load_gather(dord_v, [dsel])
                for j in range(jparts):
                    s = pl.ds(j * L, L)
                    ev = plsc.load_gather(emb_v, [ib[dd, s]])
                    outb[dd, s] = ev + (ord0b + obc[j] * dordb)

        def pair_body(q2, _):
            for slot in range(2):
                q = q2 * 2 + slot

                in_copy(q, slot).wait()

                @pl.when(q2 > 0)
                def _():
                    out_copy(q - 2, slot).wait()

                compute(q, slot)
                out_copy(q, slot).start()

                @pl.when(q2 < nq // 2 - 1)
                def _():
                    in_copy(q + 2, slot).start()

            return 0

        lax.fori_loop(0, nq // 2, pair_body, 0)
        out_copy(nq - 2, 0).wait()
        out_copy(nq - 1, 1).wait()

    return k(idx_t, pi_flat, emb_flat, ord_flat)


def kernel(edge_index, edge_data, node_data, node_num, edge_num, padded_index,
           padding_mask, padded_node_mask, padded_edge_mask, emb_table,
           order_table):
    b, t = padded_node_mask.shape
    d = node_data.shape[-1]

    idx_t = node_data.T
    pi_flat = padded_index.transpose(0, 2, 1).reshape(-1)
    emb_flat = emb_table.reshape(-1)
    ord_flat = order_table.reshape(-1)

    out3 = _tokenizer_sc(idx_t, pi_flat, emb_flat, ord_flat,
                         b_total=b, t_len=t)
    return out3.transpose(0, 2, 1)
